# baseline (speedup 1.0000x reference)
import jax
import jax.numpy as jnp
from jax import lax
from jax.experimental import pallas as pl
from jax.experimental.pallas import tpu as pltpu

H = 32
W = 32
F = 384
HW = H * W
K = 8
CPK = (2 * F) // K


def _pos_body(row_ref, col_ref, out_hbm, *rest):
    scratches = rest[:K]
    sems = rest[K]
    col_t = col_ref[...].T
    row_t = row_ref[...].T
    lane = lax.broadcasted_iota(jnp.int32, (W, HW), 1)
    sub = lax.broadcasted_iota(jnp.int32, (W, HW), 0)
    tile_mask = (lane % W == sub).astype(jnp.float32)
    rep_mask = (lane // W == sub).astype(jnp.float32)
    col_part = jnp.dot(col_t, tile_mask, precision=lax.Precision.HIGHEST,
                       preferred_element_type=jnp.float32)
    row_part = jnp.dot(row_t, rep_mask, precision=lax.Precision.HIGHEST,
                       preferred_element_type=jnp.float32)
    pos = jnp.concatenate([col_part, row_part], axis=0)
    for i in range(K):
        scratches[i][...] = pos[i * CPK:(i + 1) * CPK]
    b = out_hbm.shape[0]
    copies = [
        pltpu.make_async_copy(
            scratches[i],
            out_hbm.at[j, pl.ds(i * CPK, CPK)],
            sems.at[i, j],
        )
        for i in range(K)
        for j in range(b)
    ]
    for c in copies:
        c.start()
    for c in copies:
        c.wait()


def kernel(x, row_embed, col_embed):
    b = x.shape[0]
    out = pl.pallas_call(
        _pos_body,
        in_specs=[
            pl.BlockSpec((H, F), lambda: (0, 0)),
            pl.BlockSpec((W, F), lambda: (0, 0)),
        ],
        out_specs=pl.BlockSpec(memory_space=pl.ANY),
        out_shape=jax.ShapeDtypeStruct((b, 2 * F, HW), jnp.float32),
        scratch_shapes=[pltpu.VMEM((CPK, HW), jnp.float32) for _ in range(K)]
        + [pltpu.SemaphoreType.DMA((K, b))],
    )(row_embed, col_embed)
    return out.reshape(b, 2 * F, H, W)
